# baseline (speedup 1.0000x reference)
import jax
import jax.numpy as jnp
from jax import lax
from jax.experimental import pallas as pl
from jax.experimental.pallas import tpu as pltpu
from jax.experimental.pallas import tpu_sc as plsc

D_MODEL = 1024
NUM_CORES = 2
NUM_SUBCORES = 16
NUM_WORKERS = NUM_CORES * NUM_SUBCORES
CHUNK = 32
NBUF = 3


def _emb_body(tokens_hbm, table_hbm, out_hbm, idx_v, rows0, rows1, rows2, gsem, osem):
    b, s = tokens_hbm.shape
    b_per_w = (b * s) // NUM_WORKERS
    w_per_row = s // b_per_w
    nchunks = b_per_w // CHUNK
    wid = lax.axis_index("c") * NUM_SUBCORES + lax.axis_index("s")
    row = wid // w_per_row
    col = (wid % w_per_row) * b_per_w
    bufs = (rows0, rows1, rows2)
    pltpu.sync_copy(tokens_hbm.at[row, pl.ds(col, b_per_w)], idx_v)

    def gather(c, buf):
        return pltpu.async_copy(
            table_hbm.at[idx_v.at[pl.ds(c * CHUNK, CHUNK)]], buf, gsem
        )

    def put(c, buf):
        return pltpu.async_copy(
            buf, out_hbm.at[row, pl.ds(col + c * CHUNK, CHUNK)], osem
        )

    grefs = {}
    for c in range(min(NBUF - 1, nchunks)):
        grefs[c] = gather(c, bufs[c % NBUF])
    pending = []
    for c in range(nchunks):
        grefs.pop(c).wait()
        pending.append(put(c, bufs[c % NBUF]))
        nxt = c + NBUF - 1
        if nxt < nchunks:
            if nxt >= NBUF:
                pending.pop(0).wait()
            grefs[nxt] = gather(nxt, bufs[nxt % NBUF])
    for o in pending:
        o.wait()


@jax.jit
def _embed(tokens, weight):
    b, s = tokens.shape
    run = pl.kernel(
        _emb_body,
        out_type=jax.ShapeDtypeStruct((b, s, D_MODEL), jnp.float32),
        mesh=plsc.VectorSubcoreMesh(core_axis_name="c", subcore_axis_name="s"),
        scratch_types=[
            pltpu.VMEM(((b * s) // NUM_WORKERS,), jnp.int32),
            pltpu.VMEM((CHUNK, D_MODEL), jnp.float32),
            pltpu.VMEM((CHUNK, D_MODEL), jnp.float32),
            pltpu.VMEM((CHUNK, D_MODEL), jnp.float32),
            pltpu.SemaphoreType.DMA,
            pltpu.SemaphoreType.DMA,
        ],
    )
    return run(tokens, weight)


def kernel(tokens, weight):
    return _embed(tokens.astype(jnp.int32), weight)


# speedup vs baseline: 1.0047x; 1.0047x over previous
import jax
import jax.numpy as jnp
from jax import lax
from jax.experimental import pallas as pl
from jax.experimental.pallas import tpu as pltpu
from jax.experimental.pallas import tpu_sc as plsc

D_MODEL = 1024
NUM_CORES = 2
NUM_SUBCORES = 16
NUM_WORKERS = NUM_CORES * NUM_SUBCORES
CHUNK = 32
NBUF = 3


def _emb_body(tokens_hbm, table_hbm, out_hbm, idx_v, rows0, rows1, rows2, gsem, osem):
    b, s = tokens_hbm.shape
    b_per_w = (b * s) // NUM_WORKERS
    w_per_row = s // b_per_w
    nchunks = b_per_w // CHUNK
    wid = lax.axis_index("s") * NUM_CORES + lax.axis_index("c")
    row = wid // w_per_row
    col = (wid % w_per_row) * b_per_w
    bufs = (rows0, rows1, rows2)
    pltpu.sync_copy(tokens_hbm.at[row, pl.ds(col, b_per_w)], idx_v)

    def gather(c, buf):
        return pltpu.async_copy(
            table_hbm.at[idx_v.at[pl.ds(c * CHUNK, CHUNK)]], buf, gsem
        )

    def put(c, buf):
        return pltpu.async_copy(
            buf, out_hbm.at[row, pl.ds(col + c * CHUNK, CHUNK)], osem
        )

    grefs = {}
    for c in range(min(NBUF - 1, nchunks)):
        grefs[c] = gather(c, bufs[c % NBUF])
    pending = []
    for c in range(nchunks):
        grefs.pop(c).wait()
        pending.append(put(c, bufs[c % NBUF]))
        nxt = c + NBUF - 1
        if nxt < nchunks:
            if nxt >= NBUF:
                pending.pop(0).wait()
            grefs[nxt] = gather(nxt, bufs[nxt % NBUF])
    for o in pending:
        o.wait()


@jax.jit
def _embed(tokens, weight):
    b, s = tokens.shape
    run = pl.kernel(
        _emb_body,
        out_type=jax.ShapeDtypeStruct((b, s, D_MODEL), jnp.float32),
        mesh=plsc.VectorSubcoreMesh(core_axis_name="c", subcore_axis_name="s"),
        scratch_types=[
            pltpu.VMEM(((b * s) // NUM_WORKERS,), jnp.int32),
            pltpu.VMEM((CHUNK, D_MODEL), jnp.float32),
            pltpu.VMEM((CHUNK, D_MODEL), jnp.float32),
            pltpu.VMEM((CHUNK, D_MODEL), jnp.float32),
            pltpu.SemaphoreType.DMA,
            pltpu.SemaphoreType.DMA,
        ],
    )
    return run(tokens, weight)


def kernel(tokens, weight):
    return _embed(tokens.astype(jnp.int32), weight)
